# baseline (speedup 1.0000x reference)
import functools

import jax
import jax.numpy as jnp
from jax import lax
from jax.experimental import pallas as pl
from jax.experimental.pallas import tpu as pltpu
from jax.experimental.pallas import tpu_sc as plsc


@functools.cache
def _build(N: int, M: int, V: int, D: int):
    info = plsc.get_sparse_core_info()
    NC, NS = info.num_cores, info.num_subcores
    NW = NC * NS
    B = N * M
    assert D % NW == 0
    d_per_w = D // NW
    assert B % 16 == 0
    mesh = plsc.VectorSubcoreMesh(core_axis_name="c", subcore_axis_name="s")

    @functools.partial(
        pl.kernel,
        mesh=mesh,
        out_type=jax.ShapeDtypeStruct((N, D, M), jnp.float32),
        scratch_types=[
            pltpu.VMEM((B,), jnp.int32),
            pltpu.VMEM((d_per_w * V,), jnp.float32),
            pltpu.VMEM((d_per_w * B,), jnp.float32),
            pltpu.SemaphoreType.DMA,
            pltpu.SemaphoreType.DMA,
        ],
        compiler_params=pltpu.CompilerParams(use_tc_tiling_on_sc=True,
                                             needs_layout_passes=False),
    )
    def gather_kernel(idx_hbm, wt_hbm, out_hbm, idx_v, wrow_v, orow_v,
                      ssem, osem):
        wid = lax.axis_index("s") * NC + lax.axis_index("c")
        d0 = wid * d_per_w
        wcopies = [
            pltpu.async_copy(wt_hbm.at[d0 + dd], wrow_v.at[pl.ds(dd * V, V)],
                             ssem)
            for dd in range(d_per_w)
        ]
        icopies = [
            pltpu.async_copy(idx_hbm.at[i], idx_v.at[pl.ds(i * M, M)], ssem)
            for i in range(N)
        ]
        for c in wcopies:
            c.wait()
        outs = []
        for i in range(N):
            icopies[i].wait()

            @plsc.parallel_loop(i * M, (i + 1) * M, step=16, unroll=8)
            def _gather(b):
                iv = idx_v[pl.ds(b, 16)]
                for dd in range(d_per_w):
                    orow_v[pl.ds(dd * B + b, 16)] = plsc.load_gather(
                        wrow_v, [iv + dd * V])

            outs += [
                pltpu.async_copy(orow_v.at[pl.ds(dd * B + i * M, M)],
                                 out_hbm.at[i, d0 + dd], osem)
                for dd in range(d_per_w)
            ]
        for o in outs:
            o.wait()

    return gather_kernel


@jax.jit
def kernel(embed_id, weight):
    n, m = embed_id.shape
    v, d = weight.shape
    out_t = _build(n, m, v, d)(embed_id.astype(jnp.int32), weight.T)
    return jnp.transpose(out_t, (0, 2, 1))


# speedup vs baseline: 1.0846x; 1.0846x over previous
import functools

import jax
import jax.numpy as jnp
from jax import lax
from jax.experimental import pallas as pl
from jax.experimental.pallas import tpu as pltpu
from jax.experimental.pallas import tpu_sc as plsc


@functools.cache
def _build(N: int, M: int, V: int, D: int):
    info = plsc.get_sparse_core_info()
    NC, NS = info.num_cores, info.num_subcores
    NW = NC * NS
    B = N * M
    assert D % NW == 0
    d_per_w = D // NW
    assert B % 16 == 0
    mesh = plsc.VectorSubcoreMesh(core_axis_name="c", subcore_axis_name="s")

    @functools.partial(
        pl.kernel,
        mesh=mesh,
        out_type=jax.ShapeDtypeStruct((N, D, M), jnp.float32),
        scratch_types=[
            pltpu.VMEM((B,), jnp.int32),
            pltpu.VMEM((d_per_w * V,), jnp.float32),
            pltpu.VMEM((d_per_w * B,), jnp.float32),
            pltpu.SemaphoreType.DMA,
            pltpu.SemaphoreType.DMA,
        ],
        compiler_params=pltpu.CompilerParams(use_tc_tiling_on_sc=True,
                                             needs_layout_passes=False),
    )
    def gather_kernel(idx_hbm, wt_hbm, out_hbm, idx_v, wrow_v, orow_v,
                      ssem, osem):
        wid = lax.axis_index("s") * NC + lax.axis_index("c")
        d0 = wid * d_per_w
        wcopies = [
            pltpu.async_copy(wt_hbm.at[d0 + dd], wrow_v.at[pl.ds(dd * V, V)],
                             ssem)
            for dd in range(d_per_w)
        ]
        icopies = [
            pltpu.async_copy(idx_hbm.at[i], idx_v.at[pl.ds(i * M, M)], ssem)
            for i in range(N)
        ]
        for c in wcopies:
            c.wait()
        outs = []
        half = N // 2
        for h in range(2):
            for i in range(h * half, (h + 1) * half):
                icopies[i].wait()

            @plsc.parallel_loop(h * half * M, (h + 1) * half * M,
                                step=16, unroll=8)
            def _gather(b):
                iv = idx_v[pl.ds(b, 16)]
                for dd in range(d_per_w):
                    orow_v[pl.ds(dd * B + b, 16)] = plsc.load_gather(
                        wrow_v, [iv + dd * V])

            outs += [
                pltpu.async_copy(orow_v.at[pl.ds(dd * B + i * M, M)],
                                 out_hbm.at[i, d0 + dd], osem)
                for dd in range(d_per_w)
                for i in range(h * half, (h + 1) * half)
            ]
        for o in outs:
            o.wait()

    return gather_kernel


@jax.jit
def kernel(embed_id, weight):
    n, m = embed_id.shape
    v, d = weight.shape
    out_t = _build(n, m, v, d)(embed_id.astype(jnp.int32), weight.T)
    return jnp.transpose(out_t, (0, 2, 1))
